# Initial kernel scaffold; baseline (speedup 1.0000x reference)
import functools

import jax
import jax.numpy as jnp
from jax import lax
from jax.experimental import pallas as pl
from jax.experimental.pallas import tpu as pltpu
from jax.experimental.pallas import tpu_sc as plsc

_EPS = 1e-5
_H = 8
_PACK = 16
_LN = _PACK * _H


def _tables_body(x_ref, wa_ref, wb_ref, ta_ref, tb_ref):
    x = x_ref[:]
    ta_ref[:] = jnp.dot(x, wa_ref[:], preferred_element_type=jnp.float32,
                        precision=lax.Precision.HIGHEST)
    tb_ref[:] = jnp.dot(x, wb_ref[:], preferred_element_type=jnp.float32,
                        precision=lax.Precision.HIGHEST)


def _make_tables(x, w1a, w1b):
    n = x.shape[0]
    return pl.pallas_call(
        _tables_body,
        out_shape=[jax.ShapeDtypeStruct((n, _H), jnp.float32)] * 2,
    )(x, w1a, w1b)


def _sc_gather(ta, tb, start, end):
    info = plsc.get_sparse_core_info()
    nc, ns = info.num_cores, info.num_subcores
    nw = nc * ns
    e = start.shape[0]
    per_w = e // nw
    ch = 2000
    n_ch = per_w // ch
    assert per_w % ch == 0 and per_w % 8 == 0 and ch % 8 == 0

    mesh = plsc.VectorSubcoreMesh(core_axis_name="c", subcore_axis_name="s")

    @functools.partial(
        pl.kernel,
        mesh=mesh,
        out_type=[jax.ShapeDtypeStruct((e, _H), jnp.float32)] * 2,
        scratch_types=[
            pltpu.VMEM((ch,), jnp.int32),
            pltpu.VMEM((ch,), jnp.int32),
            pltpu.VMEM((ch, _H), jnp.float32),
            pltpu.VMEM((ch, _H), jnp.float32),
            pltpu.SemaphoreType.DMA,
            pltpu.SemaphoreType.DMA,
        ],
    )
    def gather_kernel(ta_hbm, tb_hbm, s_hbm, e_hbm, oa_hbm, ob_hbm,
                      sidx, eidx, arows, brows, sema, semb):
        wid = lax.axis_index("s") * nc + lax.axis_index("c")
        base = wid * per_w
        for k in range(n_ch):
            off = base + k * ch
            pltpu.sync_copy(s_hbm.at[pl.ds(off, ch)], sidx)
            pltpu.sync_copy(e_hbm.at[pl.ds(off, ch)], eidx)
            cpa = pltpu.async_copy(ta_hbm.at[sidx], arows, sema)
            cpb = pltpu.async_copy(tb_hbm.at[eidx], brows, semb)
            cpa.wait()
            cpb.wait()
            pltpu.sync_copy(arows, oa_hbm.at[pl.ds(off, ch)])
            pltpu.sync_copy(brows, ob_hbm.at[pl.ds(off, ch)])

    return gather_kernel(ta, tb, start, end)


def _mlp_body(sa_ref, sb_ref, m_ref, w2_ref, w3_ref, w4_ref, vec_ref,
              b4_ref, o_ref):
    f32 = jnp.float32
    m = m_ref[:]

    def dot(a, b):
        return jnp.dot(a, b, preferred_element_type=f32,
                       precision=lax.Precision.HIGHEST)

    def ln_tanh(y, gi, bi):
        mu = dot(y, m)
        d = y - mu
        v = dot(d * d, m)
        return jnp.tanh(d * lax.rsqrt(v + _EPS) * vec_ref[gi:gi + 1, :]
                        + vec_ref[bi:bi + 1, :])

    s = sa_ref[:] + sb_ref[:] + vec_ref[0:1, :]
    h = ln_tanh(s, 1, 2)
    h = ln_tanh(dot(h, w2_ref[:]) + vec_ref[3:4, :], 4, 5)
    h = ln_tanh(dot(h, w3_ref[:]) + vec_ref[6:7, :], 7, 8)
    o_ref[:] = dot(h, w4_ref[:]) + b4_ref[0]


def _mlp(sa2, sb2, m, w2bd, w3bd, w4bd, vecs, b4):
    r2, ln = sa2.shape
    br = 1000
    assert r2 % br == 0
    const = lambda i: (0, 0)
    return pl.pallas_call(
        _mlp_body,
        grid=(r2 // br,),
        in_specs=[
            pl.BlockSpec((br, ln), lambda i: (i, 0)),
            pl.BlockSpec((br, ln), lambda i: (i, 0)),
            pl.BlockSpec((ln, ln), const),
            pl.BlockSpec((ln, ln), const),
            pl.BlockSpec((ln, ln), const),
            pl.BlockSpec((ln, _PACK), const),
            pl.BlockSpec((16, ln), const),
            pl.BlockSpec(memory_space=pltpu.SMEM),
        ],
        out_specs=pl.BlockSpec((br, _PACK), lambda i: (i, 0)),
        out_shape=jax.ShapeDtypeStruct((r2, _PACK), jnp.float32),
    )(sa2, sb2, m, w2bd, w3bd, w4bd, vecs, b4)


def kernel(x, edge_index, W1, b1, g1, be1, W2, b2, g2, be2, W3, b3, g3, be3,
           W4, b4):
    n, d = x.shape
    e = edge_index.shape[1]

    ta, tb = _make_tables(x, W1[:d], W1[d:])
    sa, sb = _sc_gather(ta, tb, edge_index[0], edge_index[1])

    sa2 = sa.reshape(e // _PACK, _LN)
    sb2 = sb.reshape(e // _PACK, _LN)

    eye = jnp.eye(_PACK, dtype=jnp.float32)
    m = jnp.kron(eye, jnp.full((_H, _H), 1.0 / _H, jnp.float32))
    w2bd = jnp.kron(eye, W2)
    w3bd = jnp.kron(eye, W3)
    w4bd = jnp.kron(eye, W4)
    rows = [jnp.tile(v, _PACK) for v in (b1, g1, be1, b2, g2, be2, b3, g3, be3)]
    vecs = jnp.stack(rows + [jnp.zeros(_LN, jnp.float32)] * 7)

    out = _mlp(sa2, sb2, m, w2bd, w3bd, w4bd, vecs, b4)
    return out.reshape(e)


# speedup vs baseline: 5.2459x; 5.2459x over previous
import functools

import jax
import jax.numpy as jnp
from jax import lax
from jax.experimental import pallas as pl
from jax.experimental.pallas import tpu as pltpu
from jax.experimental.pallas import tpu_sc as plsc

_EPS = 1e-5
_H = 8
_PACK = 16
_LN = _PACK * _H


def _tables_body(x_ref, wa_ref, wb_ref, ta_ref, tb_ref):
    x = x_ref[:]
    ta_ref[:] = jnp.dot(x, wa_ref[:], preferred_element_type=jnp.float32,
                        precision=lax.Precision.HIGHEST)
    tb_ref[:] = jnp.dot(x, wb_ref[:], preferred_element_type=jnp.float32,
                        precision=lax.Precision.HIGHEST)


def _make_tables(x, w1a, w1b):
    n = x.shape[0]
    return pl.pallas_call(
        _tables_body,
        out_shape=[jax.ShapeDtypeStruct((n, _H), jnp.float32)] * 2,
    )(x, w1a, w1b)


def _sc_gather(ta, tb, start, end):
    info = plsc.get_sparse_core_info()
    nc, ns = info.num_cores, info.num_subcores
    nw = nc * ns
    e = start.shape[0]
    per_w = e // nw
    ch = 2000
    n_ch = per_w // ch
    assert per_w % ch == 0 and per_w % 8 == 0 and ch % 8 == 0

    mesh = plsc.VectorSubcoreMesh(core_axis_name="c", subcore_axis_name="s")

    @functools.partial(
        pl.kernel,
        mesh=mesh,
        compiler_params=pltpu.CompilerParams(use_tc_tiling_on_sc=False),
        out_type=[jax.ShapeDtypeStruct((e, _H), jnp.float32)] * 2,
        scratch_types=[
            pltpu.VMEM((ch,), jnp.int32),
            pltpu.VMEM((ch,), jnp.int32),
            pltpu.VMEM((ch, _H), jnp.float32),
            pltpu.VMEM((ch, _H), jnp.float32),
            pltpu.SemaphoreType.DMA,
            pltpu.SemaphoreType.DMA,
        ],
    )
    def gather_kernel(ta_hbm, tb_hbm, s_hbm, e_hbm, oa_hbm, ob_hbm,
                      sidx, eidx, arows, brows, sema, semb):
        wid = lax.axis_index("s") * nc + lax.axis_index("c")
        base = wid * per_w
        for k in range(n_ch):
            off = base + k * ch
            pltpu.sync_copy(s_hbm.at[pl.ds(off, ch)], sidx)
            pltpu.sync_copy(e_hbm.at[pl.ds(off, ch)], eidx)
            cpa = pltpu.async_copy(ta_hbm.at[sidx], arows, sema)
            cpb = pltpu.async_copy(tb_hbm.at[eidx], brows, semb)
            cpa.wait()
            cpb.wait()
            pltpu.sync_copy(arows, oa_hbm.at[pl.ds(off, ch)])
            pltpu.sync_copy(brows, ob_hbm.at[pl.ds(off, ch)])

    return gather_kernel(ta, tb, start, end)


def _mlp_body(sa_ref, sb_ref, m_ref, w2_ref, w3_ref, w4_ref, vec_ref,
              b4_ref, o_ref):
    f32 = jnp.float32
    m = m_ref[:]

    def dot(a, b):
        return jnp.dot(a, b, preferred_element_type=f32,
                       precision=lax.Precision.HIGHEST)

    def ln_tanh(y, gi, bi):
        mu = dot(y, m)
        d = y - mu
        v = dot(d * d, m)
        return jnp.tanh(d * lax.rsqrt(v + _EPS) * vec_ref[gi:gi + 1, :]
                        + vec_ref[bi:bi + 1, :])

    s = sa_ref[:] + sb_ref[:] + vec_ref[0:1, :]
    h = ln_tanh(s, 1, 2)
    h = ln_tanh(dot(h, w2_ref[:]) + vec_ref[3:4, :], 4, 5)
    h = ln_tanh(dot(h, w3_ref[:]) + vec_ref[6:7, :], 7, 8)
    o_ref[:] = dot(h, w4_ref[:]) + b4_ref[0]


def _mlp(sa2, sb2, m, w2bd, w3bd, w4bd, vecs, b4):
    r2, ln = sa2.shape
    br = 1000
    assert r2 % br == 0
    const = lambda i: (0, 0)
    return pl.pallas_call(
        _mlp_body,
        grid=(r2 // br,),
        in_specs=[
            pl.BlockSpec((br, ln), lambda i: (i, 0)),
            pl.BlockSpec((br, ln), lambda i: (i, 0)),
            pl.BlockSpec((ln, ln), const),
            pl.BlockSpec((ln, ln), const),
            pl.BlockSpec((ln, ln), const),
            pl.BlockSpec((ln, _PACK), const),
            pl.BlockSpec((16, ln), const),
            pl.BlockSpec(memory_space=pltpu.SMEM),
        ],
        out_specs=pl.BlockSpec((br, _PACK), lambda i: (i, 0)),
        out_shape=jax.ShapeDtypeStruct((r2, _PACK), jnp.float32),
    )(sa2, sb2, m, w2bd, w3bd, w4bd, vecs, b4)


def kernel(x, edge_index, W1, b1, g1, be1, W2, b2, g2, be2, W3, b3, g3, be3,
           W4, b4):
    n, d = x.shape
    e = edge_index.shape[1]

    ta, tb = _make_tables(x, W1[:d], W1[d:])
    sa, sb = _sc_gather(ta, tb, edge_index[0], edge_index[1])

    sa2 = sa.reshape(e // _PACK, _LN)
    sb2 = sb.reshape(e // _PACK, _LN)

    eye = jnp.eye(_PACK, dtype=jnp.float32)
    m = jnp.kron(eye, jnp.full((_H, _H), 1.0 / _H, jnp.float32))
    w2bd = jnp.kron(eye, W2)
    w3bd = jnp.kron(eye, W3)
    w4bd = jnp.kron(eye, W4)
    rows = [jnp.tile(v, _PACK) for v in (b1, g1, be1, b2, g2, be2, b3, g3, be3)]
    vecs = jnp.stack(rows + [jnp.zeros(_LN, jnp.float32)] * 7)

    out = _mlp(sa2, sb2, m, w2bd, w3bd, w4bd, vecs, b4)
    return out.reshape(e)
